# baseline (device time: 11135 ns/iter reference)
import jax
import jax.numpy as jnp
from jax import lax
from jax.experimental import pallas as pl
from jax.experimental.pallas import tpu as pltpu

N_DEV = 8


CHUNK = 256


def kernel(x):
    m_per, n = x.shape
    n_chunks = m_per // CHUNK

    def body(x_ref, out_ref, gather_ref, chunk_buf, copy_sems,
             send_sems, recv_sems):
        my_pos = lax.axis_index("i")

        barrier_sem = pltpu.get_barrier_semaphore()
        for d in range(1, N_DEV):
            peer = lax.rem(my_pos + d, N_DEV)
            pl.semaphore_signal(
                barrier_sem, inc=1,
                device_id=(peer,), device_id_type=pl.DeviceIdType.MESH,
            )

        def copy_chunk(c, slot):
            return pltpu.make_async_copy(
                x_ref.at[pl.ds(c * CHUNK, CHUNK), :],
                chunk_buf.at[slot],
                copy_sems.at[slot],
            )

        copies = [copy_chunk(0, 0)]
        copies[0].start()
        acc = jnp.zeros((1, n), jnp.float32)
        for c in range(n_chunks):
            if c + 1 < n_chunks:
                nxt = copy_chunk(c + 1, (c + 1) % 2)
                nxt.start()
                copies.append(nxt)
            copies[c].wait()
            acc = acc + jnp.sum(chunk_buf[c % 2], axis=0, keepdims=True)
        gather_ref[pl.ds(0, 1), :] = acc

        pl.semaphore_wait(barrier_sem, N_DEV - 1)

        sends = []
        for d in range(1, N_DEV):
            peer = lax.rem(my_pos + d, N_DEV)
            rdma = pltpu.make_async_remote_copy(
                src_ref=gather_ref.at[pl.ds(0, 1)],
                dst_ref=gather_ref.at[pl.ds(d, 1)],
                send_sem=send_sems.at[d - 1],
                recv_sem=recv_sems.at[d - 1],
                device_id=(peer,),
                device_id_type=pl.DeviceIdType.MESH,
            )
            rdma.start()
            sends.append(rdma)

        for rdma in sends:
            rdma.wait_recv()
        for rdma in sends:
            rdma.wait_send()

        total = jnp.sum(gather_ref[:, :], axis=0, keepdims=True)
        out_ref[:, :] = total * (1.0 / (N_DEV * m_per))

    return pl.pallas_call(
        body,
        out_shape=jax.ShapeDtypeStruct((1, n), jnp.float32),
        in_specs=[pl.BlockSpec(memory_space=pl.ANY)],
        out_specs=pl.BlockSpec(memory_space=pltpu.VMEM),
        scratch_shapes=[
            pltpu.VMEM((N_DEV, n), jnp.float32),
            pltpu.VMEM((2, CHUNK, n), jnp.float32),
            pltpu.SemaphoreType.DMA((2,)),
            pltpu.SemaphoreType.DMA((N_DEV - 1,)),
            pltpu.SemaphoreType.DMA((N_DEV - 1,)),
        ],
        compiler_params=pltpu.CompilerParams(collective_id=0),
    )(x)
